# Initial kernel scaffold; baseline (speedup 1.0000x reference)
import jax
import jax.numpy as jnp
from jax.experimental import pallas as pl


def kernel(normal_features, extreme_features, edge_index, batch, conv_W, att_src, att_dst, conv_b, fc1_W, fc1_b, fc_W, fc_b):
    raise NotImplementedError("write your pallas kernel here")


# speedup vs baseline: 4.7703x; 4.7703x over previous
import functools

import jax
import jax.numpy as jnp
from jax.experimental import pallas as pl
from jax.experimental.pallas import tpu as pltpu

N = 10000
E = 320000
IN = 128
HID = 256
H = 2
NG = 64
BLK = 1000


def _dense_kernel(x_ref, convW_ref, fc1W_ref, V_ref, fc1b_ref,
                  xp_ref, mlp_ref, a_ref):
    xb = x_ref[:]
    xp_ref[:] = jnp.dot(xb, convW_ref[:], preferred_element_type=jnp.float32)
    mlp_ref[:] = jnp.maximum(
        jnp.dot(xb, fc1W_ref[:], preferred_element_type=jnp.float32)
        + fc1b_ref[:], 0.0)
    a_ref[:] = jnp.dot(xb, V_ref[:], preferred_element_type=jnp.float32)


def _pool_kernel(agg_ref, mlp_ref, batch_ref, convb_ref, fcW_ref, fcb_ref,
                 out_ref, sums_ref, cnt_ref):
    i = pl.program_id(0)
    nsteps = pl.num_programs(0)
    gnn = agg_ref[:] * 0.5 + convb_ref[:]
    comb = mlp_ref[:] + jnp.maximum(gnn, 0.0)
    gid = jax.lax.broadcasted_iota(jnp.int32, (BLK, NG), 1)
    ob = (batch_ref[:] == gid).astype(jnp.float32)
    psum = jax.lax.dot_general(ob, comb, (((0,), (0,)), ((), ())),
                               preferred_element_type=jnp.float32)
    pcnt = jax.lax.dot_general(ob, jnp.ones((BLK, 8), jnp.float32),
                               (((0,), (0,)), ((), ())),
                               preferred_element_type=jnp.float32)

    @pl.when(i == 0)
    def _():
        sums_ref[:] = psum
        cnt_ref[:] = pcnt

    @pl.when(i > 0)
    def _():
        sums_ref[:] = sums_ref[:] + psum
        cnt_ref[:] = cnt_ref[:] + pcnt

    @pl.when(i == nsteps - 1)
    def _():
        gf = sums_ref[:] / jnp.maximum(cnt_ref[:, 0:1], 1.0)
        logit = jnp.dot(gf, fcW_ref[:], preferred_element_type=jnp.float32)
        out_ref[:] = jax.nn.sigmoid(logit + fcb_ref[:])


def kernel(normal_features, extreme_features, edge_index, batch,
           conv_W, att_src, att_dst, conv_b, fc1_W, fc1_b, fc_W, fc_b):
    x = jnp.concatenate([normal_features, extreme_features], axis=1)
    K = 2 * IN

    Wr = conv_W.reshape(K, H, HID)
    V_src = jnp.einsum('khc,hc->kh', Wr, att_src)
    V_dst = jnp.einsum('khc,hc->kh', Wr, att_dst)
    V = jnp.concatenate([V_src, V_dst, jnp.zeros((K, 4), jnp.float32)], axis=1)

    nblk = N // BLK
    xp, mlp, a = pl.pallas_call(
        _dense_kernel,
        grid=(nblk,),
        in_specs=[
            pl.BlockSpec((BLK, K), lambda i: (i, 0)),
            pl.BlockSpec((K, H * HID), lambda i: (0, 0)),
            pl.BlockSpec((K, HID), lambda i: (0, 0)),
            pl.BlockSpec((K, 8), lambda i: (0, 0)),
            pl.BlockSpec((1, HID), lambda i: (0, 0)),
        ],
        out_specs=[
            pl.BlockSpec((BLK, H * HID), lambda i: (i, 0)),
            pl.BlockSpec((BLK, HID), lambda i: (i, 0)),
            pl.BlockSpec((BLK, 8), lambda i: (i, 0)),
        ],
        out_shape=[
            jax.ShapeDtypeStruct((N, H * HID), jnp.float32),
            jax.ShapeDtypeStruct((N, HID), jnp.float32),
            jax.ShapeDtypeStruct((N, 8), jnp.float32),
        ],
    )(x, conv_W, fc1_W, V, fc1_b.reshape(1, HID))

    loops = jnp.arange(N, dtype=edge_index.dtype)
    src = jnp.concatenate([edge_index[0], loops])
    dst = jnp.concatenate([edge_index[1], loops])
    a_s = a[:, 0:H]
    a_d = a[:, H:2 * H]
    e = a_s[src] + a_d[dst]
    e = jnp.where(e >= 0, e, 0.2 * e)
    emax = jax.ops.segment_max(e, dst, num_segments=N)
    emax = jnp.where(jnp.isfinite(emax), emax, 0.0)
    ex = jnp.exp(e - emax[dst])
    denom = jax.ops.segment_sum(ex, dst, num_segments=N)
    alpha = ex / (denom[dst] + 1e-16)
    xps = xp[src]
    msg = xps[:, :HID] * alpha[:, 0:1] + xps[:, HID:] * alpha[:, 1:2]
    agg2 = jax.ops.segment_sum(msg, dst, num_segments=N)

    out = pl.pallas_call(
        _pool_kernel,
        grid=(nblk,),
        in_specs=[
            pl.BlockSpec((BLK, HID), lambda i: (i, 0)),
            pl.BlockSpec((BLK, HID), lambda i: (i, 0)),
            pl.BlockSpec((BLK, 1), lambda i: (i, 0)),
            pl.BlockSpec((1, HID), lambda i: (0, 0)),
            pl.BlockSpec((HID, 1), lambda i: (0, 0)),
            pl.BlockSpec((1, 1), lambda i: (0, 0)),
        ],
        out_specs=pl.BlockSpec((NG, 1), lambda i: (0, 0)),
        out_shape=jax.ShapeDtypeStruct((NG, 1), jnp.float32),
        scratch_shapes=[
            pltpu.VMEM((NG, HID), jnp.float32),
            pltpu.VMEM((NG, 8), jnp.float32),
        ],
    )(agg2, mlp, batch.reshape(N, 1), conv_b.reshape(1, HID),
      fc_W, fc_b.reshape(1, 1))
    return out
